# Initial kernel scaffold; baseline (speedup 1.0000x reference)
import jax
import jax.numpy as jnp
from jax.experimental import pallas as pl


def kernel(x):
    raise NotImplementedError("write your pallas kernel here")


# speedup vs baseline: 4.6984x; 4.6984x over previous
import functools

import jax
import jax.numpy as jnp
from jax.experimental import pallas as pl

K = 64
ROWS_PER_BLOCK = 8


def _keys_from_f32(x):
    bits = jax.lax.bitcast_convert_type(x, jnp.uint32)
    neg = bits >= jnp.uint32(0x80000000)
    return jnp.where(neg, ~bits, bits | jnp.uint32(0x80000000))


def _f32_from_key(u):
    pos = u >= jnp.uint32(0x80000000)
    bits = jnp.where(pos, u ^ jnp.uint32(0x80000000), ~u)
    return jax.lax.bitcast_convert_type(bits, jnp.float32)


def _block_kernel(x_ref, o_ref):
    x = x_ref[...]
    v = jnp.maximum(x, 0.0)
    mag = jnp.sum(v, axis=1, keepdims=True)

    u = _keys_from_f32(x)

    lo = jnp.zeros((x.shape[0], 1), jnp.uint32)
    hi = jnp.full((x.shape[0], 1), 0xFFFFFFFF, jnp.uint32)

    def body(_, carry):
        lo, hi = carry
        mid = lo + ((hi - lo) >> 1)
        cnt = jnp.sum((u > mid).astype(jnp.int32), axis=1, keepdims=True)
        go_up = cnt >= K
        lo = jnp.where(go_up, mid + 1, lo)
        hi = jnp.where(go_up, hi, mid)
        return lo, hi

    lo, hi = jax.lax.fori_loop(0, 32, body, (lo, hi))
    t_key = lo
    t = _f32_from_key(t_key)

    above = u > t_key
    c_above = jnp.sum(above.astype(jnp.float32), axis=1, keepdims=True)
    s_above = jnp.sum(jnp.where(above, x, 0.0), axis=1, keepdims=True)
    magk = s_above + t * (K - c_above)

    scale = jnp.where(mag > 0.0, magk / mag, 0.0)
    o_ref[...] = v * scale


@jax.jit
def kernel(x):
    b, n = x.shape
    grid = b // ROWS_PER_BLOCK
    return pl.pallas_call(
        _block_kernel,
        grid=(grid,),
        in_specs=[pl.BlockSpec((ROWS_PER_BLOCK, n), lambda i: (i, 0))],
        out_specs=pl.BlockSpec((ROWS_PER_BLOCK, n), lambda i: (i, 0)),
        out_shape=jax.ShapeDtypeStruct((b, n), jnp.float32),
    )(x)
